# baseline (speedup 1.0000x reference)
import jax
import jax.numpy as jnp
from jax import lax
from jax.experimental import pallas as pl
from jax.experimental.pallas import tpu as pltpu
from jax.experimental.pallas import tpu_sc as plsc

_B = 16384
_KD = 16
_NC = 2
_NS = 16
_NW = _NC * _NS
_BPW = _B // _NW
_NBLK = _BPW // 16


def _body(legs_hbm, votes_hbm, theta_hbm, beta_hbm, tmean_hbm, bmean_hbm,
          ov_hbm, out_hbm,
          legs_v, votes_v, trows_v, brows_v, tmean_v, bmean_v, ov_v, out_v,
          sem):
    wid = lax.axis_index("s") * _NC + lax.axis_index("c")
    base = wid * _BPW

    pltpu.sync_copy(legs_hbm.at[pl.ds(base, _BPW)], legs_v)
    pltpu.sync_copy(votes_hbm.at[pl.ds(base, _BPW)], votes_v)
    pltpu.sync_copy(ov_hbm, ov_v)

    cp_t = pltpu.async_copy(theta_hbm.at[legs_v], trows_v, sem)
    cp_b = pltpu.async_copy(beta_hbm.at[votes_v], brows_v, sem)
    cp_tm = pltpu.async_copy(tmean_hbm.at[legs_v], tmean_v, sem)
    cp_bm = pltpu.async_copy(bmean_hbm.at[votes_v], bmean_v, sem)
    cp_t.wait()
    cp_b.wait()
    cp_tm.wait()
    cp_bm.wait()

    iota = lax.iota(jnp.int32, 16)
    ov = ov_v[...]
    for j in range(_NBLK):
        rbase = j * 16
        row_idx = rbase + iota
        acc = tmean_v[pl.ds(rbase, 16)] + bmean_v[pl.ds(rbase, 16)] + ov
        for k in range(_KD):
            col = jnp.full((16,), k, jnp.int32)
            t = plsc.load_gather(trows_v, [row_idx, col])
            b = plsc.load_gather(brows_v, [row_idx, col])
            acc = acc + t * b
        out_v[pl.ds(rbase, 16)] = acc

    pltpu.sync_copy(out_v, out_hbm.at[pl.ds(base, _BPW)])


def kernel(legs, votes, theta, beta, theta_mean, beta_mean, overall_mean):
    ov16 = jnp.broadcast_to(overall_mean, (16,))
    mesh = plsc.VectorSubcoreMesh(core_axis_name="c", subcore_axis_name="s")
    f = pl.kernel(
        _body,
        out_type=jax.ShapeDtypeStruct((_B,), jnp.float32),
        mesh=mesh,
        compiler_params=pltpu.CompilerParams(needs_layout_passes=False,
                                              use_tc_tiling_on_sc=False),
        scratch_types=[
            pltpu.VMEM((_BPW,), jnp.int32),
            pltpu.VMEM((_BPW,), jnp.int32),
            pltpu.VMEM((_BPW, _KD), jnp.float32),
            pltpu.VMEM((_BPW, _KD), jnp.float32),
            pltpu.VMEM((_BPW,), jnp.float32),
            pltpu.VMEM((_BPW,), jnp.float32),
            pltpu.VMEM((16,), jnp.float32),
            pltpu.VMEM((_BPW,), jnp.float32),
            pltpu.SemaphoreType.DMA,
        ],
    )
    return f(legs, votes, theta, beta, theta_mean, beta_mean, ov16)
